# baseline (speedup 1.0000x reference)
import functools

import jax
import jax.numpy as jnp
from jax.experimental import pallas as pl

_M = 1024
_TK = 256


def _assign_body(x_ref, ct_ref, c2h_ref, o_ref):
    M, TK = _M, _TK
    K = ct_ref.shape[1]
    xt = x_ref[...]
    acc_v = jnp.full((M, TK), -jnp.inf, jnp.float32)
    acc_c = jnp.zeros((M, TK), jnp.int32)
    for j in range(K // TK):
        ctj = ct_ref[:, j * TK:(j + 1) * TK]
        xc = jax.lax.dot_general(
            xt, ctj, (((1,), (0,)), ((), ())),
            preferred_element_type=jnp.float32)
        s = xc - c2h_ref[:, j * TK:(j + 1) * TK]
        pred = s > acc_v
        acc_v = jnp.where(pred, s, acc_v)
        acc_c = jnp.where(pred, j, acc_c)
    m = jnp.max(acc_v, axis=1, keepdims=True)
    lane = jax.lax.broadcasted_iota(jnp.int32, (M, TK), 1)
    full_idx = acc_c * TK + lane
    cand = jnp.where(acc_v == m, full_idx, K)
    o_ref[0, 0, :] = jnp.min(cand, axis=1)


@functools.partial(jax.jit, static_argnames=("interpret",))
def _assign(x, centers, interpret=False):
    B, S, D = x.shape
    K = centers.shape[0]
    N = B * S
    M = _M
    G = N // M
    xf = x.reshape(N, D)
    ct = centers.T
    c2h = 0.5 * jnp.sum(centers * centers, axis=1)[None, :]
    out = pl.pallas_call(
        _assign_body,
        grid=(G,),
        in_specs=[
            pl.BlockSpec((M, D), lambda i: (i, 0)),
            pl.BlockSpec((D, K), lambda i: (0, 0)),
            pl.BlockSpec((1, K), lambda i: (0, 0)),
        ],
        out_specs=pl.BlockSpec((1, 1, M), lambda i: (i, 0, 0)),
        out_shape=jax.ShapeDtypeStruct((G, 1, M), jnp.int32),
        interpret=interpret,
    )(xf, ct, c2h)
    return out.reshape(B, S).astype(jnp.int64)


def kernel(x, centers):
    return _assign(x, centers)


# speedup vs baseline: 1.3903x; 1.3903x over previous
import functools

import jax
import jax.numpy as jnp
from jax.experimental import pallas as pl

_M = 1024
_TK = 256


def _assign_body(x_ref, ct_ref, c2h_ref, o_ref):
    M, TK = _M, _TK
    K = ct_ref.shape[1]
    xt = x_ref[...]
    acc_v = jnp.full((M, TK), -jnp.inf, jnp.float32)
    acc_c = jnp.zeros((M, TK), jnp.int32)
    for j in range(K // TK):
        ctj = ct_ref[:, j * TK:(j + 1) * TK]
        xc = jax.lax.dot_general(
            xt, ctj, (((1,), (0,)), ((), ())),
            preferred_element_type=jnp.float32)
        s = xc - c2h_ref[:, j * TK:(j + 1) * TK]
        pred = s > acc_v
        acc_v = jnp.where(pred, s, acc_v)
        acc_c = jnp.where(pred, j, acc_c)
    m = jnp.max(acc_v, axis=1, keepdims=True)
    lane = jax.lax.broadcasted_iota(jnp.int32, (M, TK), 1)
    full_idx = acc_c * TK + lane
    cand = jnp.where(acc_v == m, full_idx, K)
    o_ref[...] = jnp.min(cand, axis=1, keepdims=True)


@functools.partial(jax.jit, static_argnames=("interpret",))
def _assign(x, centers, interpret=False):
    B, S, D = x.shape
    K = centers.shape[0]
    N = B * S
    M = _M
    G = N // M
    xf = x.reshape(N, D)
    ct = centers.T
    c2h = 0.5 * jnp.sum(centers * centers, axis=1)[None, :]
    out = pl.pallas_call(
        _assign_body,
        grid=(G,),
        in_specs=[
            pl.BlockSpec((M, D), lambda i: (i, 0)),
            pl.BlockSpec((D, K), lambda i: (0, 0)),
            pl.BlockSpec((1, K), lambda i: (0, 0)),
        ],
        out_specs=pl.BlockSpec((M, 1), lambda i: (i, 0)),
        out_shape=jax.ShapeDtypeStruct((N, 1), jnp.int32),
        interpret=interpret,
    )(xf, ct, c2h)
    return out.reshape(B, S).astype(jnp.int64)


def kernel(x, centers):
    return _assign(x, centers)
